# baseline (speedup 1.0000x reference)
import jax
import jax.numpy as jnp
from jax.experimental import pallas as pl
from jax.experimental.pallas import tpu as pltpu

B = 4096
D = 1024
O = 1024
E = 8
H = 128
TOP_K = 2

TB = 1024


def _moe_block_kernel(x_ref, wg_ref, w1_ref, w2_ref, out_ref, w1s, w2s):
    i = pl.program_id(0)

    @pl.when(i == 0)
    def _cast_weights():
        for e in range(E):
            w1s[:, e * H:(e + 1) * H] = w1_ref[e].astype(jnp.bfloat16)
        w2s[...] = w2_ref[...].astype(jnp.bfloat16)

    x = x_ref[...]
    logits = jnp.dot(x, wg_ref[...], preferred_element_type=jnp.float32)

    eidx = jax.lax.broadcasted_iota(jnp.int32, logits.shape, 1)
    m1 = jnp.max(logits, axis=1, keepdims=True)
    i1 = jnp.min(jnp.where(logits == m1, eidx, E), axis=1, keepdims=True)
    masked = jnp.where(eidx == i1, -jnp.inf, logits)
    m2 = jnp.max(masked, axis=1, keepdims=True)
    i2 = jnp.min(jnp.where(masked == m2, eidx, E), axis=1, keepdims=True)
    p1 = 1.0 / (1.0 + jnp.exp(m2 - m1))
    p2 = 1.0 - p1
    comb = jnp.where(eidx == i1, p1, 0.0) + jnp.where(eidx == i2, p2, 0.0)

    xb = x.astype(jnp.bfloat16)
    h_all = jnp.dot(xb, w1s[...], preferred_element_type=jnp.float32)
    h_all = jnp.maximum(h_all, 0.0).astype(jnp.bfloat16)
    comb_bf = comb.astype(jnp.bfloat16)
    hs = []
    for e in range(E):
        hs.append(h_all[:, e * H:(e + 1) * H] * comb_bf[:, e:e + 1])
    hcat = jnp.concatenate(hs, axis=1)
    out_ref[...] = jnp.dot(hcat, w2s[...], preferred_element_type=jnp.float32)


@jax.jit
def kernel(x, Wg, bg, W1, b1, W2, b2):
    return pl.pallas_call(
        _moe_block_kernel,
        grid=(B // TB,),
        in_specs=[
            pl.BlockSpec((TB, D), lambda i: (i, 0)),
            pl.BlockSpec((D, E), lambda i: (0, 0)),
            pl.BlockSpec((E, D, H), lambda i: (0, 0, 0)),
            pl.BlockSpec((E * H, O), lambda i: (0, 0)),
        ],
        out_specs=pl.BlockSpec((TB, O), lambda i: (i, 0)),
        out_shape=jax.ShapeDtypeStruct((B, O), jnp.float32),
        scratch_shapes=[
            pltpu.VMEM((D, E * H), jnp.bfloat16),
            pltpu.VMEM((E * H, O), jnp.bfloat16),
        ],
    )(x, Wg, W1, W2.reshape(E * H, O))
